# baseline (speedup 1.0000x reference)
import functools

import jax
import jax.numpy as jnp
from jax import lax
from jax.experimental import pallas as pl
from jax.experimental.pallas import tpu as pltpu
from jax.experimental.pallas import tpu_sc as plsc

B = 16384
D = 16
L = 16
NC = 2
NS = 16
NW = NC * NS
BPW = B // NW
NCHUNK = BPW // L


def _shuffle(x, perm):
    return lax.gather(
        x, perm[:, None],
        dimension_numbers=lax.GatherDimensionNumbers(
            offset_dims=(), collapsed_slice_dims=(0,), start_index_map=(0,)),
        slice_sizes=(1,),
        mode=lax.GatherScatterMode.PROMISE_IN_BOUNDS)


def _sc_body(uid_hbm, iid_hbm, p_hbm, q_hbm, ub_hbm, ib_hbm, bu_hbm, cu_hbm,
             bi_hbm, ci_hbm, uw_hbm, iw_hbm, xu_hbm, xi_hbm, ua_hbm, ia_hbm,
             out_hbm,
             idx_u, idx_i, pu_v, qi_v, ub_v, ib_v, bu_v, cu_v, bi_v, ci_v,
             uw_v, iw_v, xu_v, xi_v, ua_v, ia_v, out_v, sem):
    wid = lax.axis_index("s") * NC + lax.axis_index("c")
    base = wid * BPW

    pltpu.sync_copy(uid_hbm.at[pl.ds(base, BPW)], idx_u)
    pltpu.sync_copy(iid_hbm.at[pl.ds(base, BPW)], idx_i)

    def _fix(c, carry):
        o = c * L
        u = idx_u[pl.ds(o, L)]
        idx_u[pl.ds(o, L)] = jnp.maximum(u - 1, 0)
        v = idx_i[pl.ds(o, L)]
        idx_i[pl.ds(o, L)] = jnp.maximum(v - 1, 0)
        return carry

    lax.fori_loop(0, NCHUNK, _fix, 0)

    copies = [
        pltpu.async_copy(p_hbm.at[idx_u], pu_v, sem),
        pltpu.async_copy(q_hbm.at[idx_i], qi_v, sem),
        pltpu.async_copy(ub_hbm.at[idx_u], ub_v, sem),
        pltpu.async_copy(ib_hbm.at[idx_i], ib_v, sem),
        pltpu.async_copy(bu_hbm.at[idx_u], bu_v, sem),
        pltpu.async_copy(cu_hbm.at[idx_u], cu_v, sem),
        pltpu.async_copy(bi_hbm.at[idx_i], bi_v, sem),
        pltpu.async_copy(ci_hbm.at[idx_i], ci_v, sem),
        pltpu.async_copy(uw_hbm.at[idx_u], uw_v, sem),
        pltpu.async_copy(iw_hbm.at[idx_i], iw_v, sem),
        pltpu.async_copy(xu_hbm.at[idx_u], xu_v, sem),
        pltpu.async_copy(xi_hbm.at[idx_i], xi_v, sem),
        pltpu.async_copy(ua_hbm.at[idx_u], ua_v, sem),
        pltpu.async_copy(ia_hbm.at[idx_i], ia_v, sem),
    ]
    for cp in copies:
        cp.wait()

    iota = lax.iota(jnp.int32, L)
    perms = [jnp.bitwise_xor(iota, k) for k in (8, 4, 2, 1)]

    def _compute(c, carry):
        o = c * L
        acc = jnp.zeros((L,), jnp.float32)
        for r in range(L):
            row = o + r
            prod = pu_v[row, :] * qi_v[row, :]
            for perm in perms:
                prod = prod + _shuffle(prod, perm)
            acc = jnp.where(iota == r, prod, acc)
        ub = ub_v[pl.ds(o, L)]
        ib = ib_v[pl.ds(o, L)]
        bu = bu_v[pl.ds(o, L)]
        cu = cu_v[pl.ds(o, L)]
        bi = bi_v[pl.ds(o, L)]
        ci = ci_v[pl.ds(o, L)]
        uw = uw_v[pl.ds(o, L)]
        iw = iw_v[pl.ds(o, L)]
        xu = xu_v[pl.ds(o, L)]
        xi = xi_v[pl.ds(o, L)]
        ua = ua_v[pl.ds(o, L)]
        ia = ia_v[pl.ds(o, L)]
        mf_out = ub + ib + acc
        yu = bu * xu + cu
        yi = bi * xi + ci
        lr = uw * yu + iw * yi
        out_v[pl.ds(o, L)] = mf_out * 0.8 + lr * 0.2 + (ua + ia)
        return carry

    lax.fori_loop(0, NCHUNK, _compute, 0)

    pltpu.sync_copy(out_v, out_hbm.at[pl.ds(base, BPW)])


_sc_call = functools.partial(
    pl.kernel,
    out_type=jax.ShapeDtypeStruct((B,), jnp.float32),
    mesh=plsc.VectorSubcoreMesh(core_axis_name="c", subcore_axis_name="s"),
    compiler_params=pltpu.CompilerParams(use_tc_tiling_on_sc=False),
    scratch_types=[
        pltpu.VMEM((BPW,), jnp.int32),
        pltpu.VMEM((BPW,), jnp.int32),
        pltpu.VMEM((BPW, D), jnp.float32),
        pltpu.VMEM((BPW, D), jnp.float32),
        pltpu.VMEM((BPW,), jnp.float32),
        pltpu.VMEM((BPW,), jnp.float32),
        pltpu.VMEM((BPW,), jnp.float32),
        pltpu.VMEM((BPW,), jnp.float32),
        pltpu.VMEM((BPW,), jnp.float32),
        pltpu.VMEM((BPW,), jnp.float32),
        pltpu.VMEM((BPW,), jnp.float32),
        pltpu.VMEM((BPW,), jnp.float32),
        pltpu.VMEM((BPW,), jnp.float32),
        pltpu.VMEM((BPW,), jnp.float32),
        pltpu.VMEM((BPW,), jnp.float32),
        pltpu.VMEM((BPW,), jnp.float32),
        pltpu.VMEM((BPW,), jnp.float32),
        pltpu.SemaphoreType.DMA,
    ],
)(_sc_body)


def kernel(sparse_inputs, p, q, user_bias, item_bias, beta_u, bias_u,
           beta_i, bias_i, user_weight, item_weight, user_hs, item_hs,
           u_avg, i_avg):
    uid_raw = sparse_inputs[:, 0]
    iid_raw = sparse_inputs[:, 1]
    out = _sc_call(
        uid_raw, iid_raw, p, q,
        user_bias.reshape(-1), item_bias.reshape(-1),
        beta_u.reshape(-1), bias_u.reshape(-1),
        beta_i.reshape(-1), bias_i.reshape(-1),
        user_weight.reshape(-1), item_weight.reshape(-1),
        user_hs, item_hs, u_avg, i_avg,
    )
    return out.reshape(B, 1)
